# speedup vs baseline: 1.5161x; 1.5161x over previous
import jax
import jax.numpy as jnp
from jax.experimental import pallas as pl

N_TOKENS = 16384
IN_FEATURES = 4096
N_EXPERTS = 64
TOP_K = 8
BM = 512


def _gate_kernel(x_ref, wt_ref, w_out_ref, i_out_ref):
    logits = jnp.dot(x_ref[...], wt_ref[...], preferred_element_type=jnp.float32)
    m = jnp.max(logits, axis=1, keepdims=True)
    e = jnp.exp(logits - m)
    s = e / jnp.sum(e, axis=1, keepdims=True)

    iota = jax.lax.broadcasted_iota(jnp.int32, (BM, N_EXPERTS), 1)
    for j in range(TOP_K):
        cur = jnp.max(s, axis=1, keepdims=True)
        hit = s == cur
        idx = jnp.min(jnp.where(hit, iota, N_EXPERTS), axis=1, keepdims=True)
        w_out_ref[:, j : j + 1] = cur
        i_out_ref[:, j : j + 1] = idx
        s = jnp.where(iota == idx, -1.0, s)


def kernel(x, W):
    wt = W.T
    grid = (N_TOKENS // BM,)
    weights, indices = pl.pallas_call(
        _gate_kernel,
        grid=grid,
        in_specs=[
            pl.BlockSpec((BM, IN_FEATURES), lambda i: (i, 0)),
            pl.BlockSpec((IN_FEATURES, N_EXPERTS), lambda i: (0, 0)),
        ],
        out_specs=[
            pl.BlockSpec((BM, TOP_K), lambda i: (i, 0)),
            pl.BlockSpec((BM, TOP_K), lambda i: (i, 0)),
        ],
        out_shape=[
            jax.ShapeDtypeStruct((N_TOKENS, TOP_K), jnp.float32),
            jax.ShapeDtypeStruct((N_TOKENS, TOP_K), jnp.int32),
        ],
    )(x, wt)
    return weights, indices


# speedup vs baseline: 1.6616x; 1.0960x over previous
import jax
import jax.numpy as jnp
from jax.experimental import pallas as pl

N_TOKENS = 16384
IN_FEATURES = 4096
N_EXPERTS = 64
TOP_K = 8
BM = 512


def _gate_kernel(x_ref, wt_ref, w_out_ref, i_out_ref):
    logits = jnp.dot(x_ref[...], wt_ref[...], preferred_element_type=jnp.float32)
    m = jnp.max(logits, axis=1, keepdims=True)
    e = jnp.exp(logits - m)
    s = e / jnp.sum(e, axis=1, keepdims=True)

    rev_iota = jax.lax.broadcasted_iota(jnp.int32, (BM, N_EXPERTS), 1) ^ 63
    s_bits = jax.lax.bitcast_convert_type(s, jnp.int32)
    key = (s_bits & ~jnp.int32(63)) | rev_iota
    for j in range(TOP_K):
        cur = jnp.max(key, axis=1, keepdims=True)
        w_out_ref[:, j : j + 1] = jax.lax.bitcast_convert_type(
            cur & ~jnp.int32(63), jnp.float32
        )
        i_out_ref[:, j : j + 1] = (cur & 63) ^ 63
        key = jnp.where(key == cur, jnp.int32(-1), key)


def kernel(x, W):
    wt = W.T
    grid = (N_TOKENS // BM,)
    weights, indices = pl.pallas_call(
        _gate_kernel,
        grid=grid,
        in_specs=[
            pl.BlockSpec((BM, IN_FEATURES), lambda i: (i, 0)),
            pl.BlockSpec((IN_FEATURES, N_EXPERTS), lambda i: (0, 0)),
        ],
        out_specs=[
            pl.BlockSpec((BM, TOP_K), lambda i: (i, 0)),
            pl.BlockSpec((BM, TOP_K), lambda i: (i, 0)),
        ],
        out_shape=[
            jax.ShapeDtypeStruct((N_TOKENS, TOP_K), jnp.float32),
            jax.ShapeDtypeStruct((N_TOKENS, TOP_K), jnp.int32),
        ],
    )(x, wt)
    return weights, indices


# speedup vs baseline: 1.8579x; 1.1181x over previous
import jax
import jax.numpy as jnp
from jax.experimental import pallas as pl

N_TOKENS = 16384
IN_FEATURES = 4096
N_EXPERTS = 64
TOP_K = 8
BM = 1024


def _gate_kernel(x_ref, wt_ref, w_out_ref, i_out_ref):
    logits = jnp.dot(x_ref[...], wt_ref[...], preferred_element_type=jnp.float32)
    m = jnp.max(logits, axis=1, keepdims=True)
    e = jnp.exp(logits - m)
    s = e / jnp.sum(e, axis=1, keepdims=True)

    rev_iota = jax.lax.broadcasted_iota(jnp.int32, (BM, N_EXPERTS), 1) ^ 63
    s_bits = jax.lax.bitcast_convert_type(s, jnp.int32)
    key = (s_bits & ~jnp.int32(63)) | rev_iota
    for j in range(TOP_K):
        cur = jnp.max(key, axis=1, keepdims=True)
        w_out_ref[:, j : j + 1] = jax.lax.bitcast_convert_type(
            cur & ~jnp.int32(63), jnp.float32
        )
        i_out_ref[:, j : j + 1] = (cur & 63) ^ 63
        key = jnp.where(key == cur, jnp.int32(-1), key)


def kernel(x, W):
    wt = W.T
    grid = (N_TOKENS // BM,)
    weights, indices = pl.pallas_call(
        _gate_kernel,
        grid=grid,
        in_specs=[
            pl.BlockSpec((BM, IN_FEATURES), lambda i: (i, 0)),
            pl.BlockSpec((IN_FEATURES, N_EXPERTS), lambda i: (0, 0)),
        ],
        out_specs=[
            pl.BlockSpec((BM, TOP_K), lambda i: (i, 0)),
            pl.BlockSpec((BM, TOP_K), lambda i: (i, 0)),
        ],
        out_shape=[
            jax.ShapeDtypeStruct((N_TOKENS, TOP_K), jnp.float32),
            jax.ShapeDtypeStruct((N_TOKENS, TOP_K), jnp.int32),
        ],
    )(x, wt)
    return weights, indices
